# Initial kernel scaffold; baseline (speedup 1.0000x reference)
import jax
import jax.numpy as jnp
from jax.experimental import pallas as pl


def kernel(feat0, w1, gamma, beta, mean, var, w2, b2):
    raise NotImplementedError("write your pallas kernel here")


# speedup vs baseline: 2.1563x; 2.1563x over previous
import functools

import numpy as np
import jax
import jax.numpy as jnp
from jax.experimental import pallas as pl
from jax.experimental.pallas import tpu as pltpu


def _head_kernel(feat_ref, w1_ref, shift_ref, w2_ref, b2_ref, pred_ref,
                 *, height, width, cin):
    hw = height * width
    x = feat_ref[0].astype(jnp.bfloat16)
    col = jax.lax.broadcasted_iota(jnp.int32, (1, hw), 1)
    wpos = jax.lax.rem(col, width)
    hpos = jax.lax.div(col, width)

    acc = None
    for ky in range(3):
        for kx in range(3):
            s = (ky - 1) * width + (kx - 1)
            xt = x if s == 0 else jnp.roll(x, -s, axis=1)
            conds = []
            if ky == 0:
                conds.append(hpos >= 1)
            if ky == 2:
                conds.append(hpos <= height - 2)
            if kx == 0:
                conds.append(wpos >= 1)
            if kx == 2:
                conds.append(wpos <= width - 2)
            if conds:
                m = conds[0]
                for extra in conds[1:]:
                    m = jnp.logical_and(m, extra)
                xt = jnp.where(m, xt, jnp.bfloat16(0))
            k = ky * 3 + kx
            part = jnp.dot(w1_ref[:, k * cin:(k + 1) * cin], xt,
                           preferred_element_type=jnp.float32)
            acc = part if acc is None else acc + part

    mid = jnp.maximum(acc + shift_ref[...], 0.0).astype(jnp.bfloat16)
    pred_ref[0] = jnp.dot(w2_ref[...], mid,
                          preferred_element_type=jnp.float32) + b2_ref[...]


def _resize_kernel(pred_ref, ah_ref, awt_ref, out_ref, *, n_planes, width):
    hm = jnp.dot(ah_ref[...], pred_ref[...],
                 preferred_element_type=jnp.float32)
    for p in range(n_planes):
        out_ref[p] = jnp.dot(hm[:, p * width:(p + 1) * width], awt_ref[...],
                             preferred_element_type=jnp.float32)


def _interp_weights(n_in, n_out):
    mat = np.zeros((n_out, n_in), np.float32)
    if n_in == 1 or n_out == 1:
        mat[:, 0] = 1.0
        return mat
    pos = np.arange(n_out, dtype=np.float64) * (n_in - 1) / (n_out - 1)
    lo = np.clip(np.floor(pos).astype(np.int64), 0, n_in - 1)
    hi = np.minimum(lo + 1, n_in - 1)
    frac = (pos - lo).astype(np.float32)
    rows = np.arange(n_out)
    mat[rows, lo] += 1.0 - frac
    mat[rows, hi] += frac
    return mat


def _forward(feat, w1, gamma, beta, mean, var, w2, b2,
             out_size=(512, 512), eps=1e-5, row_tile=128):
    B, Cin, H, W = feat.shape
    Cmid = w1.shape[0]
    C = w2.shape[0]
    OH, OW = out_size
    HW = H * W
    P = B * C

    scale = gamma * jax.lax.rsqrt(var + eps)
    shift = (beta - mean * scale).reshape(Cmid, 1)
    w1t = (jnp.transpose(w1, (0, 2, 3, 1)).reshape(Cmid, 9 * Cin)
           * scale[:, None]).astype(jnp.bfloat16)
    w2t = w2.reshape(C, Cmid).astype(jnp.bfloat16)
    b2c = b2.reshape(C, 1)
    feat3 = feat.reshape(B, Cin, HW)

    head_cost = pl.CostEstimate(
        flops=2 * Cmid * 9 * Cin * HW * B + 2 * C * Cmid * HW * B,
        transcendentals=0,
        bytes_accessed=4 * Cin * HW * B + 2 * Cmid * 9 * Cin * B
                       + 4 * C * HW * B)

    pred = pl.pallas_call(
        functools.partial(_head_kernel, height=H, width=W, cin=Cin),
        out_shape=jax.ShapeDtypeStruct((B, C, HW), jnp.float32),
        grid=(B,),
        in_specs=[
            pl.BlockSpec((1, Cin, HW), lambda b: (b, 0, 0)),
            pl.BlockSpec((Cmid, 9 * Cin), lambda b: (0, 0)),
            pl.BlockSpec((Cmid, 1), lambda b: (0, 0)),
            pl.BlockSpec((C, Cmid), lambda b: (0, 0)),
            pl.BlockSpec((C, 1), lambda b: (0, 0)),
        ],
        out_specs=pl.BlockSpec((1, C, HW), lambda b: (b, 0, 0)),
        compiler_params=pltpu.CompilerParams(
            dimension_semantics=("parallel",),
            vmem_limit_bytes=64 * 1024 * 1024),
        cost_estimate=head_cost,
    )(feat3, w1t, shift, w2t, b2c)

    pred_hpw = (pred.reshape(B, C, H, W).transpose(2, 0, 1, 3)
                .reshape(H, P * W))
    ah = jnp.asarray(_interp_weights(H, OH))
    awt = jnp.asarray(_interp_weights(W, OW).T)

    TH = row_tile
    assert OH % TH == 0, (OH, TH)
    n_steps = OH // TH
    resize_cost = pl.CostEstimate(
        flops=2 * OH * H * P * W + 2 * P * OH * W * OW,
        transcendentals=0,
        bytes_accessed=4 * (P * OH * OW + H * P * W + OH * H + W * OW))

    out = pl.pallas_call(
        functools.partial(_resize_kernel, n_planes=P, width=W),
        out_shape=jax.ShapeDtypeStruct((P, OH, OW), jnp.float32),
        grid=(n_steps,),
        in_specs=[
            pl.BlockSpec((H, P * W), lambda r: (0, 0)),
            pl.BlockSpec((TH, H), lambda r: (r, 0)),
            pl.BlockSpec((W, OW), lambda r: (0, 0)),
        ],
        out_specs=pl.BlockSpec((P, TH, OW), lambda r: (0, r, 0)),
        compiler_params=pltpu.CompilerParams(
            dimension_semantics=("parallel",),
            vmem_limit_bytes=64 * 1024 * 1024),
        cost_estimate=resize_cost,
    )(pred_hpw, ah, awt)

    return {'blob_pred': out.reshape(B, C, OH, OW)}


def kernel(feat0, w1, gamma, beta, mean, var, w2, b2):
    return _forward(feat0, w1, gamma, beta, mean, var, w2, b2)


# speedup vs baseline: 2.2959x; 1.0647x over previous
import functools

import numpy as np
import jax
import jax.numpy as jnp
from jax.experimental import pallas as pl
from jax.experimental.pallas import tpu as pltpu


def _head_kernel(feat_ref, w1_ref, shift_ref, w2_ref, b2_ref, pred_ref,
                 mid_ref, *, height, width, cin, n_csplit):
    j = pl.program_id(1)
    cslab = w1_ref.shape[0]
    hw = height * width
    x = feat_ref[0].astype(jnp.bfloat16)
    col = jax.lax.broadcasted_iota(jnp.int32, (1, hw), 1)
    wpos = jax.lax.rem(col, width)
    hpos = jax.lax.div(col, width)

    acc = None
    for ky in range(3):
        for kx in range(3):
            s = (ky - 1) * width + (kx - 1)
            xt = x if s == 0 else jnp.roll(x, -s, axis=1)
            conds = []
            if ky == 0:
                conds.append(hpos >= 1)
            if ky == 2:
                conds.append(hpos <= height - 2)
            if kx == 0:
                conds.append(wpos >= 1)
            if kx == 2:
                conds.append(wpos <= width - 2)
            if conds:
                m = conds[0]
                for extra in conds[1:]:
                    m = jnp.logical_and(m, extra)
                xt = jnp.where(m, xt, jnp.bfloat16(0))
            k = ky * 3 + kx
            part = jnp.dot(w1_ref[:, k * cin:(k + 1) * cin], xt,
                           preferred_element_type=jnp.float32)
            acc = part if acc is None else acc + part

    mid = jnp.maximum(acc + shift_ref[...], 0.0).astype(jnp.bfloat16)
    mid_ref[pl.ds(j * cslab, cslab), :] = mid

    @pl.when(j == n_csplit - 1)
    def _():
        pred_ref[0] = jnp.dot(w2_ref[...], mid_ref[...],
                              preferred_element_type=jnp.float32) + b2_ref[...]


def _resize_kernel(pred_ref, ah_ref, awt_ref, out_ref, *, n_planes, width):
    hm = jnp.dot(ah_ref[...], pred_ref[...],
                 preferred_element_type=jnp.float32)
    for p in range(n_planes):
        out_ref[p] = jnp.dot(hm[:, p * width:(p + 1) * width], awt_ref[...],
                             preferred_element_type=jnp.float32)


def _interp_weights(n_in, n_out):
    mat = np.zeros((n_out, n_in), np.float32)
    if n_in == 1 or n_out == 1:
        mat[:, 0] = 1.0
        return mat
    pos = np.arange(n_out, dtype=np.float64) * (n_in - 1) / (n_out - 1)
    lo = np.clip(np.floor(pos).astype(np.int64), 0, n_in - 1)
    hi = np.minimum(lo + 1, n_in - 1)
    frac = (pos - lo).astype(np.float32)
    rows = np.arange(n_out)
    mat[rows, lo] += 1.0 - frac
    mat[rows, hi] += frac
    return mat


def _forward(feat, w1, gamma, beta, mean, var, w2, b2,
             out_size=(512, 512), eps=1e-5, row_tile=128, n_csplit=2):
    B, Cin, H, W = feat.shape
    Cmid = w1.shape[0]
    C = w2.shape[0]
    OH, OW = out_size
    HW = H * W
    P = B * C

    scale = gamma * jax.lax.rsqrt(var + eps)
    shift = (beta - mean * scale).reshape(Cmid, 1)
    w1s = (w1 * scale[:, None, None, None]).astype(jnp.bfloat16)
    w1t = jnp.transpose(w1s, (0, 2, 3, 1)).reshape(Cmid, 9 * Cin)
    w2t = w2.reshape(C, Cmid).astype(jnp.bfloat16)
    b2c = b2.reshape(C, 1)
    feat3 = feat.reshape(B, Cin, HW)

    head_cost = pl.CostEstimate(
        flops=2 * Cmid * 9 * Cin * HW * B + 2 * C * Cmid * HW * B,
        transcendentals=0,
        bytes_accessed=4 * Cin * HW * B + 2 * Cmid * 9 * Cin * B
                       + 4 * C * HW * B)

    assert Cmid % n_csplit == 0
    cslab = Cmid // n_csplit
    pred = pl.pallas_call(
        functools.partial(_head_kernel, height=H, width=W, cin=Cin,
                          n_csplit=n_csplit),
        out_shape=jax.ShapeDtypeStruct((B, C, HW), jnp.float32),
        grid=(B, n_csplit),
        in_specs=[
            pl.BlockSpec((1, Cin, HW), lambda b, j: (b, 0, 0)),
            pl.BlockSpec((cslab, 9 * Cin), lambda b, j: (j, 0)),
            pl.BlockSpec((cslab, 1), lambda b, j: (j, 0)),
            pl.BlockSpec((C, Cmid), lambda b, j: (0, 0)),
            pl.BlockSpec((C, 1), lambda b, j: (0, 0)),
        ],
        out_specs=pl.BlockSpec((1, C, HW), lambda b, j: (b, 0, 0)),
        scratch_shapes=[pltpu.VMEM((Cmid, HW), jnp.bfloat16)],
        compiler_params=pltpu.CompilerParams(
            dimension_semantics=("parallel", "arbitrary"),
            vmem_limit_bytes=64 * 1024 * 1024),
        cost_estimate=head_cost,
    )(feat3, w1t, shift, w2t, b2c)

    pred_hpw = (pred.reshape(B, C, H, W).transpose(2, 0, 1, 3)
                .reshape(H, P * W))
    ah = jnp.asarray(_interp_weights(H, OH))
    awt = jnp.asarray(_interp_weights(W, OW).T)

    TH = row_tile
    assert OH % TH == 0, (OH, TH)
    n_steps = OH // TH
    resize_cost = pl.CostEstimate(
        flops=2 * OH * H * P * W + 2 * P * OH * W * OW,
        transcendentals=0,
        bytes_accessed=4 * (P * OH * OW + H * P * W + OH * H + W * OW))

    out = pl.pallas_call(
        functools.partial(_resize_kernel, n_planes=P, width=W),
        out_shape=jax.ShapeDtypeStruct((P, OH, OW), jnp.float32),
        grid=(n_steps,),
        in_specs=[
            pl.BlockSpec((H, P * W), lambda r: (0, 0)),
            pl.BlockSpec((TH, H), lambda r: (r, 0)),
            pl.BlockSpec((W, OW), lambda r: (0, 0)),
        ],
        out_specs=pl.BlockSpec((P, TH, OW), lambda r: (0, r, 0)),
        compiler_params=pltpu.CompilerParams(
            dimension_semantics=("parallel",),
            vmem_limit_bytes=64 * 1024 * 1024),
        cost_estimate=resize_cost,
    )(pred_hpw, ah, awt)

    return {'blob_pred': out.reshape(B, C, OH, OW)}


def kernel(feat0, w1, gamma, beta, mean, var, w2, b2):
    return _forward(feat0, w1, gamma, beta, mean, var, w2, b2)


# speedup vs baseline: 2.8550x; 1.2435x over previous
import functools

import numpy as np
import jax
import jax.numpy as jnp
from jax.experimental import pallas as pl
from jax.experimental.pallas import tpu as pltpu


def _head_kernel(feat_ref, w1_ref, scale_ref, shift_ref, w2_ref, b2_ref,
                 pred_ref, mid_ref, *, height, width, cin, n_csplit):
    j = pl.program_id(1)
    cslab = w1_ref.shape[0]
    hw = height * width
    x = feat_ref[0].astype(jnp.bfloat16)
    col = jax.lax.broadcasted_iota(jnp.int32, (1, hw), 1)
    wpos = jax.lax.rem(col, width)
    hpos = jax.lax.div(col, width)

    acc = None
    for ky in range(3):
        for kx in range(3):
            s = (ky - 1) * width + (kx - 1)
            xt = x if s == 0 else jnp.roll(x, -s, axis=1)
            conds = []
            if ky == 0:
                conds.append(hpos >= 1)
            if ky == 2:
                conds.append(hpos <= height - 2)
            if kx == 0:
                conds.append(wpos >= 1)
            if kx == 2:
                conds.append(wpos <= width - 2)
            if conds:
                m = conds[0]
                for extra in conds[1:]:
                    m = jnp.logical_and(m, extra)
                xt = jnp.where(m, xt, jnp.bfloat16(0))
            k = ky * 3 + kx
            part = jnp.dot(w1_ref[:, k * cin:(k + 1) * cin], xt,
                           preferred_element_type=jnp.float32)
            acc = part if acc is None else acc + part

    mid = jnp.maximum(acc * scale_ref[...] + shift_ref[...],
                      0.0).astype(jnp.bfloat16)
    mid_ref[pl.ds(j * cslab, cslab), :] = mid

    @pl.when(j == n_csplit - 1)
    def _():
        pred_ref[0] = jnp.dot(w2_ref[...], mid_ref[...],
                              preferred_element_type=jnp.float32) + b2_ref[...]


def _resize_kernel(pred_ref, ah_ref, awt_ref, out_ref, *, n_planes, width):
    hm = jnp.dot(ah_ref[...], pred_ref[...],
                 preferred_element_type=jnp.float32)
    for p in range(n_planes):
        out_ref[p] = jnp.dot(hm[:, p * width:(p + 1) * width], awt_ref[...],
                             preferred_element_type=jnp.float32)


def _interp_weights(n_in, n_out):
    mat = np.zeros((n_out, n_in), np.float32)
    if n_in == 1 or n_out == 1:
        mat[:, 0] = 1.0
        return mat
    pos = np.arange(n_out, dtype=np.float64) * (n_in - 1) / (n_out - 1)
    lo = np.clip(np.floor(pos).astype(np.int64), 0, n_in - 1)
    hi = np.minimum(lo + 1, n_in - 1)
    frac = (pos - lo).astype(np.float32)
    rows = np.arange(n_out)
    mat[rows, lo] += 1.0 - frac
    mat[rows, hi] += frac
    return mat


def _forward(feat, w1, gamma, beta, mean, var, w2, b2,
             out_size=(512, 512), eps=1e-5, row_tile=128, n_csplit=4):
    B, Cin, H, W = feat.shape
    Cmid = w1.shape[0]
    C = w2.shape[0]
    OH, OW = out_size
    HW = H * W
    P = B * C

    scale = gamma * jax.lax.rsqrt(var + eps)
    shift = (beta - mean * scale).reshape(Cmid, 1)
    scale2 = scale.reshape(Cmid, 1)
    w1t = (jnp.transpose(w1.astype(jnp.bfloat16), (0, 2, 3, 1))
           .reshape(Cmid, 9 * Cin))
    w2t = w2.reshape(C, Cmid).astype(jnp.bfloat16)
    b2c = b2.reshape(C, 1)
    feat3 = feat.reshape(B, Cin, HW)

    head_cost = pl.CostEstimate(
        flops=2 * Cmid * 9 * Cin * HW * B + 2 * C * Cmid * HW * B,
        transcendentals=0,
        bytes_accessed=4 * Cin * HW * B + 2 * Cmid * 9 * Cin * B
                       + 4 * C * HW * B)

    assert Cmid % n_csplit == 0
    cslab = Cmid // n_csplit
    pred = pl.pallas_call(
        functools.partial(_head_kernel, height=H, width=W, cin=Cin,
                          n_csplit=n_csplit),
        out_shape=jax.ShapeDtypeStruct((B, C, HW), jnp.float32),
        grid=(B, n_csplit),
        in_specs=[
            pl.BlockSpec((1, Cin, HW), lambda b, j: (b, 0, 0)),
            pl.BlockSpec((cslab, 9 * Cin), lambda b, j: (j, 0)),
            pl.BlockSpec((cslab, 1), lambda b, j: (j, 0)),
            pl.BlockSpec((cslab, 1), lambda b, j: (j, 0)),
            pl.BlockSpec((C, Cmid), lambda b, j: (0, 0)),
            pl.BlockSpec((C, 1), lambda b, j: (0, 0)),
        ],
        out_specs=pl.BlockSpec((1, C, HW), lambda b, j: (b, 0, 0)),
        scratch_shapes=[pltpu.VMEM((Cmid, HW), jnp.bfloat16)],
        compiler_params=pltpu.CompilerParams(
            dimension_semantics=("parallel", "arbitrary"),
            vmem_limit_bytes=64 * 1024 * 1024),
        cost_estimate=head_cost,
    )(feat3, w1t, scale2, shift, w2t, b2c)

    pred_hpw = (pred.reshape(B, C, H, W).transpose(2, 0, 1, 3)
                .reshape(H, P * W))
    ah = jnp.asarray(_interp_weights(H, OH))
    awt = jnp.asarray(_interp_weights(W, OW).T)

    TH = row_tile
    assert OH % TH == 0, (OH, TH)
    n_steps = OH // TH
    resize_cost = pl.CostEstimate(
        flops=2 * OH * H * P * W + 2 * P * OH * W * OW,
        transcendentals=0,
        bytes_accessed=4 * (P * OH * OW + H * P * W + OH * H + W * OW))

    out = pl.pallas_call(
        functools.partial(_resize_kernel, n_planes=P, width=W),
        out_shape=jax.ShapeDtypeStruct((P, OH, OW), jnp.float32),
        grid=(n_steps,),
        in_specs=[
            pl.BlockSpec((H, P * W), lambda r: (0, 0)),
            pl.BlockSpec((TH, H), lambda r: (r, 0)),
            pl.BlockSpec((W, OW), lambda r: (0, 0)),
        ],
        out_specs=pl.BlockSpec((P, TH, OW), lambda r: (0, r, 0)),
        compiler_params=pltpu.CompilerParams(
            dimension_semantics=("parallel",),
            vmem_limit_bytes=64 * 1024 * 1024),
        cost_estimate=resize_cost,
    )(pred_hpw, ah, awt)

    return {'blob_pred': out.reshape(B, C, OH, OW)}


def kernel(feat0, w1, gamma, beta, mean, var, w2, b2):
    return _forward(feat0, w1, gamma, beta, mean, var, w2, b2)


# speedup vs baseline: 2.9396x; 1.0296x over previous
import functools

import numpy as np
import jax
import jax.numpy as jnp
from jax.experimental import pallas as pl
from jax.experimental.pallas import tpu as pltpu


def _head_kernel(feat_ref, w1_ref, scale_ref, shift_ref, w2_ref, b2_ref,
                 pred_ref, patches_ref, mid_ref,
                 *, height, width, cin, n_csplit):
    j = pl.program_id(1)
    cslab = w1_ref.shape[0]
    hw = height * width

    @pl.when(j == 0)
    def _():
        x = feat_ref[0].astype(jnp.bfloat16)
        col = jax.lax.broadcasted_iota(jnp.int32, (1, hw), 1)
        wpos = jax.lax.rem(col, width)
        hpos = jax.lax.div(col, width)
        acc = None
        for ky in range(3):
            for kx in range(3):
                s = (ky - 1) * width + (kx - 1)
                xt = x if s == 0 else jnp.roll(x, -s, axis=1)
                conds = []
                if ky == 0:
                    conds.append(hpos >= 1)
                if ky == 2:
                    conds.append(hpos <= height - 2)
                if kx == 0:
                    conds.append(wpos >= 1)
                if kx == 2:
                    conds.append(wpos <= width - 2)
                if conds:
                    m = conds[0]
                    for extra in conds[1:]:
                        m = jnp.logical_and(m, extra)
                    xt = jnp.where(m, xt, jnp.bfloat16(0))
                k = ky * 3 + kx
                patches_ref[k * cin:(k + 1) * cin, :] = xt
                part = jnp.dot(w1_ref[:, k * cin:(k + 1) * cin], xt,
                               preferred_element_type=jnp.float32)
                acc = part if acc is None else acc + part
        mid = jnp.maximum(acc * scale_ref[...] + shift_ref[...],
                          0.0).astype(jnp.bfloat16)
        mid_ref[pl.ds(0, cslab), :] = mid

    @pl.when(j > 0)
    def _():
        acc = jnp.dot(w1_ref[...], patches_ref[...],
                      preferred_element_type=jnp.float32)
        mid = jnp.maximum(acc * scale_ref[...] + shift_ref[...],
                          0.0).astype(jnp.bfloat16)
        mid_ref[pl.ds(j * cslab, cslab), :] = mid

    @pl.when(j == n_csplit - 1)
    def _():
        pred_ref[0] = jnp.dot(w2_ref[...], mid_ref[...],
                              preferred_element_type=jnp.float32) + b2_ref[...]


def _resize_kernel(pred_ref, ah_ref, awt_ref, out_ref, *, n_planes, width):
    hm = jnp.dot(ah_ref[...], pred_ref[...],
                 preferred_element_type=jnp.float32)
    for p in range(n_planes):
        out_ref[p] = jnp.dot(hm[:, p * width:(p + 1) * width], awt_ref[...],
                             preferred_element_type=jnp.float32)


def _interp_weights(n_in, n_out):
    mat = np.zeros((n_out, n_in), np.float32)
    if n_in == 1 or n_out == 1:
        mat[:, 0] = 1.0
        return mat
    pos = np.arange(n_out, dtype=np.float64) * (n_in - 1) / (n_out - 1)
    lo = np.clip(np.floor(pos).astype(np.int64), 0, n_in - 1)
    hi = np.minimum(lo + 1, n_in - 1)
    frac = (pos - lo).astype(np.float32)
    rows = np.arange(n_out)
    mat[rows, lo] += 1.0 - frac
    mat[rows, hi] += frac
    return mat


def _forward(feat, w1, gamma, beta, mean, var, w2, b2,
             out_size=(512, 512), eps=1e-5, row_tile=128, n_csplit=4):
    B, Cin, H, W = feat.shape
    Cmid = w1.shape[0]
    C = w2.shape[0]
    OH, OW = out_size
    HW = H * W
    P = B * C

    scale = gamma * jax.lax.rsqrt(var + eps)
    shift = (beta - mean * scale).reshape(Cmid, 1)
    scale2 = scale.reshape(Cmid, 1)
    w1t = (jnp.transpose(w1.astype(jnp.bfloat16), (0, 2, 3, 1))
           .reshape(Cmid, 9 * Cin))
    w2t = w2.reshape(C, Cmid).astype(jnp.bfloat16)
    b2c = b2.reshape(C, 1)
    feat3 = feat.reshape(B, Cin, HW)

    head_cost = pl.CostEstimate(
        flops=2 * Cmid * 9 * Cin * HW * B + 2 * C * Cmid * HW * B,
        transcendentals=0,
        bytes_accessed=4 * Cin * HW * B + 2 * Cmid * 9 * Cin * B
                       + 4 * C * HW * B)

    assert Cmid % n_csplit == 0
    cslab = Cmid // n_csplit
    pred = pl.pallas_call(
        functools.partial(_head_kernel, height=H, width=W, cin=Cin,
                          n_csplit=n_csplit),
        out_shape=jax.ShapeDtypeStruct((B, C, HW), jnp.float32),
        grid=(B, n_csplit),
        in_specs=[
            pl.BlockSpec((1, Cin, HW), lambda b, j: (b, 0, 0)),
            pl.BlockSpec((cslab, 9 * Cin), lambda b, j: (j, 0)),
            pl.BlockSpec((cslab, 1), lambda b, j: (j, 0)),
            pl.BlockSpec((cslab, 1), lambda b, j: (j, 0)),
            pl.BlockSpec((C, Cmid), lambda b, j: (0, 0)),
            pl.BlockSpec((C, 1), lambda b, j: (0, 0)),
        ],
        out_specs=pl.BlockSpec((1, C, HW), lambda b, j: (b, 0, 0)),
        scratch_shapes=[pltpu.VMEM((9 * Cin, HW), jnp.bfloat16),
                        pltpu.VMEM((Cmid, HW), jnp.bfloat16)],
        compiler_params=pltpu.CompilerParams(
            dimension_semantics=("parallel", "arbitrary"),
            vmem_limit_bytes=64 * 1024 * 1024),
        cost_estimate=head_cost,
    )(feat3, w1t, scale2, shift, w2t, b2c)

    pred_hpw = (pred.reshape(B, C, H, W).transpose(2, 0, 1, 3)
                .reshape(H, P * W))
    ah = jnp.asarray(_interp_weights(H, OH))
    awt = jnp.asarray(_interp_weights(W, OW).T)

    TH = row_tile
    assert OH % TH == 0, (OH, TH)
    n_steps = OH // TH
    resize_cost = pl.CostEstimate(
        flops=2 * OH * H * P * W + 2 * P * OH * W * OW,
        transcendentals=0,
        bytes_accessed=4 * (P * OH * OW + H * P * W + OH * H + W * OW))

    out = pl.pallas_call(
        functools.partial(_resize_kernel, n_planes=P, width=W),
        out_shape=jax.ShapeDtypeStruct((P, OH, OW), jnp.float32),
        grid=(n_steps,),
        in_specs=[
            pl.BlockSpec((H, P * W), lambda r: (0, 0)),
            pl.BlockSpec((TH, H), lambda r: (r, 0)),
            pl.BlockSpec((W, OW), lambda r: (0, 0)),
        ],
        out_specs=pl.BlockSpec((P, TH, OW), lambda r: (0, r, 0)),
        compiler_params=pltpu.CompilerParams(
            dimension_semantics=("parallel",),
            vmem_limit_bytes=64 * 1024 * 1024),
        cost_estimate=resize_cost,
    )(pred_hpw, ah, awt)

    return {'blob_pred': out.reshape(B, C, OH, OW)}


def kernel(feat0, w1, gamma, beta, mean, var, w2, b2):
    return _forward(feat0, w1, gamma, beta, mean, var, w2, b2)
